# baseline (speedup 1.0000x reference)
import jax
import jax.numpy as jnp
from jax import lax
from jax.experimental import pallas as pl
from jax.experimental.pallas import tpu as pltpu

NUM_HEADS = 16
HEAD_DIM = 128
NUM_KV_HEADS = 4
GROUP = NUM_HEADS // NUM_KV_HEADS
SCALE = 0.08838834764831845

SEQ = 2048
BQ = 256
BK = 256
NEG_INF = float("-inf")


def _flash_kernel(q_ref, k_ref, v_ref, o_ref, s_ref):
    i = pl.program_id(0)
    tri = (lax.broadcasted_iota(jnp.int32, (BQ, BK), 0)
           >= lax.broadcasted_iota(jnp.int32, (BQ, BK), 1))

    for h in range(NUM_HEADS):
        g = h // GROUP
        q = (q_ref[:, h * HEAD_DIM:(h + 1) * HEAD_DIM] * SCALE).astype(
            jnp.bfloat16)

        def scores(j, q=q, g=g):
            k_blk = k_ref[pl.ds(j * BK, BK), g * HEAD_DIM:(g + 1) * HEAD_DIM]
            return lax.dot_general(
                q, k_blk.astype(jnp.bfloat16), (((1,), (1,)), ((), ())),
                preferred_element_type=jnp.float32,
            )

        def pass_a(j, m_acc):
            s = scores(j)
            s_ref[:, pl.ds(j * BK, BK)] = s
            return jnp.maximum(m_acc, s)

        m_acc = lax.fori_loop(0, i, pass_a, jnp.full((BQ, BK), NEG_INF,
                                                     jnp.float32))
        s_diag = jnp.where(tri, scores(i), NEG_INF)
        s_ref[:, pl.ds(i * BK, BK)] = s_diag
        m_acc = jnp.maximum(m_acc, s_diag)
        m = jnp.max(m_acc, axis=1, keepdims=True)

        def pass_b(j, carry, g=g):
            l_elem, acc = carry
            p = jnp.exp(s_ref[:, pl.ds(j * BK, BK)] - m)
            v_blk = v_ref[pl.ds(j * BK, BK), g * HEAD_DIM:(g + 1) * HEAD_DIM]
            pv = lax.dot_general(
                p.astype(jnp.bfloat16), v_blk.astype(jnp.bfloat16),
                (((1,), (0,)), ((), ())),
                preferred_element_type=jnp.float32,
            )
            return l_elem + p, acc + pv

        l0 = jnp.zeros((BQ, BK), jnp.float32)
        acc0 = jnp.zeros((BQ, HEAD_DIM), jnp.float32)
        l_elem, acc = lax.fori_loop(0, i + 1, pass_b, (l0, acc0))
        l = jnp.sum(l_elem, axis=1, keepdims=True)
        o_ref[:, h * HEAD_DIM:(h + 1) * HEAD_DIM] = acc / l


@jax.jit
def _attention(q2, k2, v2):
    return pl.pallas_call(
        _flash_kernel,
        grid=(SEQ // BQ,),
        in_specs=[
            pl.BlockSpec((BQ, NUM_HEADS * HEAD_DIM), lambda i: (i, 0)),
            pl.BlockSpec((SEQ, NUM_KV_HEADS * HEAD_DIM), lambda i: (0, 0)),
            pl.BlockSpec((SEQ, NUM_KV_HEADS * HEAD_DIM), lambda i: (0, 0)),
        ],
        out_specs=pl.BlockSpec((BQ, NUM_HEADS * HEAD_DIM), lambda i: (i, 0)),
        out_shape=jax.ShapeDtypeStruct((SEQ, NUM_HEADS * HEAD_DIM), jnp.float32),
        scratch_shapes=[pltpu.VMEM((BQ, SEQ), jnp.float32)],
        compiler_params=pltpu.CompilerParams(
            dimension_semantics=("arbitrary",),
        ),
    )(q2, k2, v2)


def kernel(q, k, v, cu_seqlens_q):
    q2 = q.reshape(SEQ, NUM_HEADS * HEAD_DIM)
    k2 = k.reshape(SEQ, NUM_KV_HEADS * HEAD_DIM)
    v2 = v.reshape(SEQ, NUM_KV_HEADS * HEAD_DIM)
    return _attention(q2, k2, v2)


# speedup vs baseline: 1.8913x; 1.8913x over previous
import jax
import jax.numpy as jnp
from jax import lax
from jax.experimental import pallas as pl
from jax.experimental.pallas import tpu as pltpu

NUM_HEADS = 16
HEAD_DIM = 128
NUM_KV_HEADS = 4
GROUP = NUM_HEADS // NUM_KV_HEADS
SCALE = 0.08838834764831845

SEQ = 2048
BQ = 256
BK = 256
CLAMP = -80.0


def _flash_kernel(q_ref, k_ref, v_ref, o_ref, kmax_ref):
    i = pl.program_id(0)
    tri = (lax.broadcasted_iota(jnp.int32, (BQ, BK), 0)
           >= lax.broadcasted_iota(jnp.int32, (BQ, BK), 1))

    @pl.when(i == 0)
    def _():
        for g in range(NUM_KV_HEADS):
            ksl = k_ref[:, g * HEAD_DIM:(g + 1) * HEAD_DIM]
            kn2 = jnp.sum(ksl * ksl, axis=1, keepdims=True)
            kmax_ref[g:g + 1, :] = jnp.broadcast_to(
                jnp.max(kn2), (1, HEAD_DIM))

    for g in range(NUM_KV_HEADS):
        kmax2 = kmax_ref[g:g + 1, 0:1]
        heads = [g * GROUP + u for u in range(GROUP)]
        qs, ms = [], []
        for h in heads:
            qf = q_ref[:, h * HEAD_DIM:(h + 1) * HEAD_DIM]
            qn2 = jnp.sum(qf * qf, axis=1, keepdims=True)
            ms.append(SCALE * jnp.sqrt(qn2 * kmax2))
            qs.append((qf * SCALE).astype(jnp.bfloat16))

        def body(j, carry, g=g, qs=qs, ms=ms):
            k_blk = k_ref[pl.ds(j * BK, BK),
                          g * HEAD_DIM:(g + 1) * HEAD_DIM].astype(jnp.bfloat16)
            v_blk = v_ref[pl.ds(j * BK, BK),
                          g * HEAD_DIM:(g + 1) * HEAD_DIM].astype(jnp.bfloat16)
            diag_mask = jnp.logical_and(j == i, jnp.logical_not(tri))
            new = []
            for u in range(GROUP):
                l_u, acc_u = carry[u]
                s = lax.dot_general(
                    qs[u], k_blk, (((1,), (1,)), ((), ())),
                    preferred_element_type=jnp.float32,
                )
                d = jnp.where(diag_mask, -jnp.inf, s - ms[u])
                p = jnp.exp(jnp.maximum(d, CLAMP))
                pv = lax.dot_general(
                    p.astype(jnp.bfloat16), v_blk, (((1,), (0,)), ((), ())),
                    preferred_element_type=jnp.float32,
                )
                new.append((l_u + jnp.sum(p, axis=1, keepdims=True),
                            acc_u + pv))
            return tuple(new)

        init = tuple((jnp.zeros((BQ, 1), jnp.float32),
                      jnp.zeros((BQ, HEAD_DIM), jnp.float32))
                     for _ in range(GROUP))
        out = lax.fori_loop(0, i + 1, body, init)
        for u, h in enumerate(heads):
            l_u, acc_u = out[u]
            o_ref[:, h * HEAD_DIM:(h + 1) * HEAD_DIM] = acc_u / l_u


@jax.jit
def _attention(q2, k2, v2):
    return pl.pallas_call(
        _flash_kernel,
        grid=(SEQ // BQ,),
        in_specs=[
            pl.BlockSpec((BQ, NUM_HEADS * HEAD_DIM), lambda i: (i, 0)),
            pl.BlockSpec((SEQ, NUM_KV_HEADS * HEAD_DIM), lambda i: (0, 0)),
            pl.BlockSpec((SEQ, NUM_KV_HEADS * HEAD_DIM), lambda i: (0, 0)),
        ],
        out_specs=pl.BlockSpec((BQ, NUM_HEADS * HEAD_DIM), lambda i: (i, 0)),
        out_shape=jax.ShapeDtypeStruct((SEQ, NUM_HEADS * HEAD_DIM), jnp.float32),
        scratch_shapes=[pltpu.VMEM((NUM_KV_HEADS, HEAD_DIM), jnp.float32)],
        compiler_params=pltpu.CompilerParams(
            dimension_semantics=("arbitrary",),
        ),
    )(q2, k2, v2)


def kernel(q, k, v, cu_seqlens_q):
    q2 = q.reshape(SEQ, NUM_HEADS * HEAD_DIM)
    k2 = k.reshape(SEQ, NUM_KV_HEADS * HEAD_DIM)
    v2 = v.reshape(SEQ, NUM_KV_HEADS * HEAD_DIM)
    return _attention(q2, k2, v2)


# speedup vs baseline: 2.7568x; 1.4576x over previous
import jax
import jax.numpy as jnp
from jax import lax
from jax.experimental import pallas as pl
from jax.experimental.pallas import tpu as pltpu

NUM_HEADS = 16
HEAD_DIM = 128
NUM_KV_HEADS = 4
GROUP = NUM_HEADS // NUM_KV_HEADS
SCALE = 0.08838834764831845
LOG2E = 1.4426950408889634
SCL2 = SCALE * LOG2E

SEQ = 2048
BQ = 512
BK = 512
CLAMP2 = -115.0


def _flash_kernel(q_ref, k_ref, v_ref, o_ref, kbf_ref, vbf_ref, kmax_ref):
    i = pl.program_id(0)
    tri = (lax.broadcasted_iota(jnp.int32, (BQ, BK), 0)
           >= lax.broadcasted_iota(jnp.int32, (BQ, BK), 1))

    @pl.when(i == 0)
    def _():
        kbf_ref[...] = k_ref[...].astype(jnp.bfloat16)
        vbf_ref[...] = v_ref[...].astype(jnp.bfloat16)
        for g in range(NUM_KV_HEADS):
            ksl = k_ref[:, g * HEAD_DIM:(g + 1) * HEAD_DIM]
            kn2 = jnp.sum(ksl * ksl, axis=1, keepdims=True)
            kmax_ref[g:g + 1, :] = jnp.broadcast_to(
                jnp.max(kn2), (1, HEAD_DIM))

    for g in range(NUM_KV_HEADS):
        kmax2 = kmax_ref[g:g + 1, 0:1]
        heads = [g * GROUP + u for u in range(GROUP)]
        qs, ms = [], []
        for h in heads:
            qf = q_ref[:, h * HEAD_DIM:(h + 1) * HEAD_DIM]
            qn2 = jnp.sum(qf * qf, axis=1, keepdims=True)
            ms.append(SCL2 * jnp.sqrt(qn2 * kmax2))
            qs.append((qf * SCL2).astype(jnp.bfloat16))

        def blocks(j, carry, masked, g=g, qs=qs, ms=ms):
            k_blk = kbf_ref[pl.ds(j * BK, BK),
                            g * HEAD_DIM:(g + 1) * HEAD_DIM]
            v_blk = vbf_ref[pl.ds(j * BK, BK),
                            g * HEAD_DIM:(g + 1) * HEAD_DIM]
            new = []
            for u in range(GROUP):
                l_u, acc_u = carry[u]
                s = lax.dot_general(
                    qs[u], k_blk, (((1,), (1,)), ((), ())),
                    preferred_element_type=jnp.float32,
                )
                d = s - ms[u]
                if masked:
                    d = jnp.where(tri, jnp.maximum(d, CLAMP2), -jnp.inf)
                p = jnp.exp2(d)
                pv = lax.dot_general(
                    p.astype(jnp.bfloat16), v_blk, (((1,), (0,)), ((), ())),
                    preferred_element_type=jnp.float32,
                )
                new.append((l_u + jnp.sum(p, axis=1, keepdims=True),
                            acc_u + pv))
            return tuple(new)

        init = tuple((jnp.zeros((BQ, 1), jnp.float32),
                      jnp.zeros((BQ, HEAD_DIM), jnp.float32))
                     for _ in range(GROUP))
        carry = lax.fori_loop(0, i, lambda j, c: blocks(j, c, False), init)
        out = blocks(i, carry, True)
        for u, h in enumerate(heads):
            l_u, acc_u = out[u]
            o_ref[:, h * HEAD_DIM:(h + 1) * HEAD_DIM] = acc_u / l_u


@jax.jit
def _attention(q2, k2, v2):
    return pl.pallas_call(
        _flash_kernel,
        grid=(SEQ // BQ,),
        in_specs=[
            pl.BlockSpec((BQ, NUM_HEADS * HEAD_DIM), lambda i: (i, 0)),
            pl.BlockSpec((SEQ, NUM_KV_HEADS * HEAD_DIM), lambda i: (0, 0)),
            pl.BlockSpec((SEQ, NUM_KV_HEADS * HEAD_DIM), lambda i: (0, 0)),
        ],
        out_specs=pl.BlockSpec((BQ, NUM_HEADS * HEAD_DIM), lambda i: (i, 0)),
        out_shape=jax.ShapeDtypeStruct((SEQ, NUM_HEADS * HEAD_DIM), jnp.float32),
        scratch_shapes=[
            pltpu.VMEM((SEQ, NUM_KV_HEADS * HEAD_DIM), jnp.bfloat16),
            pltpu.VMEM((SEQ, NUM_KV_HEADS * HEAD_DIM), jnp.bfloat16),
            pltpu.VMEM((NUM_KV_HEADS, HEAD_DIM), jnp.float32),
        ],
        compiler_params=pltpu.CompilerParams(
            dimension_semantics=("arbitrary",),
        ),
    )(q2, k2, v2)


def kernel(q, k, v, cu_seqlens_q):
    q2 = q.reshape(SEQ, NUM_HEADS * HEAD_DIM)
    k2 = k.reshape(SEQ, NUM_KV_HEADS * HEAD_DIM)
    v2 = v.reshape(SEQ, NUM_KV_HEADS * HEAD_DIM)
    return _attention(q2, k2, v2)


# speedup vs baseline: 2.7912x; 1.0125x over previous
import jax
import jax.numpy as jnp
from jax import lax
from jax.experimental import pallas as pl
from jax.experimental.pallas import tpu as pltpu

NUM_HEADS = 16
HEAD_DIM = 128
NUM_KV_HEADS = 4
GROUP = NUM_HEADS // NUM_KV_HEADS
SCALE = 0.08838834764831845
LOG2E = 1.4426950408889634
SCL2 = SCALE * LOG2E

SEQ = 2048
BQ = 512
BK = 512
MQ = GROUP * BQ
VE = HEAD_DIM * 2
CLAMP2 = -115.0


def _flash_kernel(q_ref, k_ref, v_ref, o_ref, kbf_ref, vext_ref, kmax_ref):
    i = pl.program_id(0)
    tri = ((lax.broadcasted_iota(jnp.int32, (MQ, BK), 0) & (BQ - 1))
           >= lax.broadcasted_iota(jnp.int32, (MQ, BK), 1))

    @pl.when(i == 0)
    def _():
        kbf_ref[...] = k_ref[...].astype(jnp.bfloat16)
        for g in range(NUM_KV_HEADS):
            ksl = k_ref[:, g * HEAD_DIM:(g + 1) * HEAD_DIM]
            vext_ref[:, g * VE:g * VE + HEAD_DIM] = (
                v_ref[:, g * HEAD_DIM:(g + 1) * HEAD_DIM].astype(jnp.bfloat16))
            vext_ref[:, g * VE + HEAD_DIM:(g + 1) * VE] = jnp.ones(
                (SEQ, HEAD_DIM), jnp.bfloat16)
            kn2 = jnp.sum(ksl * ksl, axis=1, keepdims=True)
            kmax_ref[g:g + 1, :] = jnp.broadcast_to(
                jnp.max(kn2), (1, HEAD_DIM))

    for g in range(NUM_KV_HEADS):
        kmax2 = kmax_ref[g:g + 1, 0:1]
        heads = [g * GROUP + u for u in range(GROUP)]
        qs, ms = [], []
        for h in heads:
            qf = q_ref[:, h * HEAD_DIM:(h + 1) * HEAD_DIM]
            qn2 = jnp.sum(qf * qf, axis=1, keepdims=True)
            ms.append(SCL2 * jnp.sqrt(qn2 * kmax2))
            qs.append((qf * SCL2).astype(jnp.bfloat16))
        q_stack = jnp.concatenate(qs, axis=0)
        m_stack = jnp.concatenate(ms, axis=0)

        def blocks(j, acc, masked, g=g, q_stack=q_stack, m_stack=m_stack):
            k_blk = kbf_ref[pl.ds(j * BK, BK),
                            g * HEAD_DIM:(g + 1) * HEAD_DIM]
            v_blk = vext_ref[pl.ds(j * BK, BK), g * VE:(g + 1) * VE]
            s = lax.dot_general(
                q_stack, k_blk, (((1,), (1,)), ((), ())),
                preferred_element_type=jnp.float32,
            )
            d = s - m_stack
            if masked:
                d = jnp.where(tri, jnp.maximum(d, CLAMP2), -jnp.inf)
            p = jnp.exp2(d).astype(jnp.bfloat16)
            return acc + lax.dot_general(
                p, v_blk, (((1,), (0,)), ((), ())),
                preferred_element_type=jnp.float32,
            )

        init = jnp.zeros((MQ, VE), jnp.float32)
        acc = lax.fori_loop(0, i, lambda j, a: blocks(j, a, False), init)
        acc = blocks(i, acc, True)
        for u, h in enumerate(heads):
            pv = acc[u * BQ:(u + 1) * BQ, :HEAD_DIM]
            l = acc[u * BQ:(u + 1) * BQ, HEAD_DIM:HEAD_DIM + 1]
            o_ref[:, h * HEAD_DIM:(h + 1) * HEAD_DIM] = pv / l


@jax.jit
def _attention(q2, k2, v2):
    return pl.pallas_call(
        _flash_kernel,
        grid=(SEQ // BQ,),
        in_specs=[
            pl.BlockSpec((BQ, NUM_HEADS * HEAD_DIM), lambda i: (i, 0)),
            pl.BlockSpec((SEQ, NUM_KV_HEADS * HEAD_DIM), lambda i: (0, 0)),
            pl.BlockSpec((SEQ, NUM_KV_HEADS * HEAD_DIM), lambda i: (0, 0)),
        ],
        out_specs=pl.BlockSpec((BQ, NUM_HEADS * HEAD_DIM), lambda i: (i, 0)),
        out_shape=jax.ShapeDtypeStruct((SEQ, NUM_HEADS * HEAD_DIM), jnp.float32),
        scratch_shapes=[
            pltpu.VMEM((SEQ, NUM_KV_HEADS * HEAD_DIM), jnp.bfloat16),
            pltpu.VMEM((SEQ, NUM_KV_HEADS * VE), jnp.bfloat16),
            pltpu.VMEM((NUM_KV_HEADS, HEAD_DIM), jnp.float32),
        ],
        compiler_params=pltpu.CompilerParams(
            dimension_semantics=("arbitrary",),
        ),
    )(q2, k2, v2)


def kernel(q, k, v, cu_seqlens_q):
    q2 = q.reshape(SEQ, NUM_HEADS * HEAD_DIM)
    k2 = k.reshape(SEQ, NUM_KV_HEADS * HEAD_DIM)
    v2 = v.reshape(SEQ, NUM_KV_HEADS * HEAD_DIM)
    return _attention(q2, k2, v2)
